# baseline (speedup 1.0000x reference)
import functools

import jax
import jax.numpy as jnp
from jax import lax
from jax.experimental import pallas as pl
from jax.experimental.pallas import tpu as pltpu
from jax.experimental.pallas import tpu_sc as plsc

B, N, NS, D, S = 4, 2048, 16, 128, 8
EPS = 1e-5
RB = 256
PB = 512
NBLK = N // RB
TW = 2 * D + 256
TWI = TW // 2
ROWS = B * N * NS
NW = 32
CH = 256


def _dotT(a, b):
    return lax.dot_general(a, b, (((1,), (1,)), ((), ())),
                           preferred_element_type=jnp.float32)


def _proj_kernel(t0_ref, ppad_ref, lin1_W_ref, bn1_gs_ref, bn1_b_ref,
                 q_W_ref, q_b_ref, k_W_ref, k_b_ref, v_W_ref, v_b_ref,
                 xq_ref, kvp_ref):
    t0 = t0_ref[...]
    x = jnp.maximum(_dotT(t0, lin1_W_ref[...]) * bn1_gs_ref[...]
                    + bn1_b_ref[...], 0.0)
    xq = _dotT(x, q_W_ref[...]) + q_b_ref[...]
    xk = _dotT(x, k_W_ref[...]) + k_b_ref[...]
    xv = _dotT(x, v_W_ref[...]) + v_b_ref[...]
    xq_ref[...] = xq
    kvp_ref[...] = jnp.concatenate(
        [xk, xv, ppad_ref[...]], axis=1).astype(jnp.bfloat16)


def _topk_kernel(p_blk_ref, p_full_ref, idx_ref):
    p_blk = p_blk_ref[...]
    p_full = p_full_ref[...]

    sq_blk = jnp.sum(p_blk * p_blk, axis=1, keepdims=True)
    sq_full = jnp.sum(p_full * p_full, axis=1, keepdims=True)
    dot = _dotT(p_blk, p_full)
    d2 = sq_blk + jnp.transpose(sq_full) - 2.0 * dot

    iota = lax.broadcasted_iota(jnp.int32, (RB, N), 1)
    key = lax.bitcast_convert_type(jnp.maximum(d2, 0.0), jnp.int32)
    key = (key & ~jnp.int32(N - 1)) | iota

    big = jnp.int32(2**31 - 1)
    cols = []
    for _ in range(NS):
        m = jnp.min(key, axis=1, keepdims=True)
        key = jnp.where(key == m, big, key)
        cols.append(m & jnp.int32(N - 1))
    idx_blk = jnp.concatenate(cols, axis=1)
    idx_ref[...] = idx_blk + pl.program_id(0) * N


def _attn_kernel(p_blk_ref, xq_ref, t0_ref, g_ref,
                 p0_W_ref, p0_b_ref, pbn_gs_ref, pbn_b_ref,
                 p2_W_ref, p2_b_ref, wbn0_gs_ref, wbn0_b_ref,
                 w2_W_ref, w2_b_ref, wbn3_gs_ref, wbn3_b_ref,
                 w5_W_ref, w5_b_ref, bn2_gs_ref, bn2_b_ref,
                 lin3_W_ref, bn3_gs_ref, bn3_b_ref,
                 mlp1_W_ref, mlp1_b_ref, mlpbn_gs_ref, mlpbn_b_ref,
                 mlp2_W_ref, out_ref):
    p_blk = p_blk_ref[...]
    xq = xq_ref[...]
    g3 = g_ref[...].astype(jnp.float32)

    w_list = []
    v_list = []
    for s in range(NS):
        g = g3[:, s, :]
        xk_s = g[:, :D]
        xv_s = g[:, D:2 * D]
        p_r = g[:, 2 * D:2 * D + 3] - p_blk
        pr = _dotT(p_r, p0_W_ref[...]) + p0_b_ref[...]
        pr = jnp.maximum(pr * pbn_gs_ref[...] + pbn_b_ref[...], 0.0)
        pr = _dotT(pr, p2_W_ref[...]) + p2_b_ref[...]
        w = xk_s - xq + pr
        w = jnp.maximum(w * wbn0_gs_ref[...] + wbn0_b_ref[...], 0.0)
        w = _dotT(w, w2_W_ref[...]) + w2_b_ref[...]
        w = jnp.maximum(w * wbn3_gs_ref[...] + wbn3_b_ref[...], 0.0)
        w = _dotT(w, w5_W_ref[...]) + w5_b_ref[...]
        w_list.append(w)
        v_list.append(xv_s + pr)

    mx = w_list[0]
    for w in w_list[1:]:
        mx = jnp.maximum(mx, w)
    e_list = [jnp.exp(w - mx) for w in w_list]
    z = e_list[0]
    for e in e_list[1:]:
        z = z + e
    rz = 1.0 / z

    c_idx = lax.broadcasted_iota(jnp.int32, (D // S, D), 1)
    g_idx = lax.broadcasted_iota(jnp.int32, (D // S, D), 0)
    expand = (jnp.remainder(c_idx, D // S) == g_idx).astype(jnp.float32)

    attn = jnp.zeros((RB, D), dtype=jnp.float32)
    for e, v in zip(e_list, v_list):
        wt = lax.dot_general(e * rz, expand, (((1,), (0,)), ((), ())),
                             preferred_element_type=jnp.float32)
        attn = attn + wt * v

    x2 = jnp.maximum(attn * bn2_gs_ref[...] + bn2_b_ref[...], 0.0)
    x3 = _dotT(x2, lin3_W_ref[...]) * bn3_gs_ref[...] + bn3_b_ref[...]
    x4 = jnp.maximum(x3 + t0_ref[...], 0.0)
    h = jnp.maximum((_dotT(x4, mlp1_W_ref[...]) + mlp1_b_ref[...])
                    * mlpbn_gs_ref[...] + mlpbn_b_ref[...], 0.0)
    y = _dotT(h, mlp2_W_ref[...])
    out_ref[...] = p_blk + y


_sc_mesh = plsc.VectorSubcoreMesh(core_axis_name="c", subcore_axis_name="s")


@functools.partial(
    pl.kernel,
    mesh=_sc_mesh,
    out_type=jax.ShapeDtypeStruct((ROWS, TWI), jnp.int32),
    scratch_types=[
        pltpu.VMEM((CH,), jnp.int32),
        pltpu.VMEM((CH, TWI), jnp.int32),
        pltpu.SemaphoreType.DMA,
    ],
)
def _sc_gather(table_hbm, idx_hbm, out_hbm, idx_v, rows_v, sem):
    wid = lax.axis_index("s") * 2 + lax.axis_index("c")
    base = wid * (ROWS // NW)
    for i in range((ROWS // NW) // CH):
        off = base + i * CH
        pltpu.sync_copy(idx_hbm.at[pl.ds(off, CH)], idx_v)
        pltpu.async_copy(table_hbm.at[idx_v], rows_v, sem).wait()
        pltpu.sync_copy(rows_v, out_hbm.at[pl.ds(off, CH)])


def kernel(pxo, transf_features, idxs, lin1_W, bn1_g, bn1_b, q_W, q_b,
           k_W, k_b, v_W, v_b, p0_W, p0_b, pbn_g, pbn_b, p2_W, p2_b,
           wbn0_g, wbn0_b, w2_W, w2_b, wbn3_g, wbn3_b, w5_W, w5_b,
           bn2_g, bn2_b, lin3_W, bn3_g, bn3_b, mlp1_W, mlp1_b,
           mlpbn_g, mlpbn_b, mlp2_W):
    n = B * N
    pxo = pxo + jnp.sum(idxs).astype(pxo.dtype)
    p_flat = pxo.reshape(n, 3)
    t0 = jnp.transpose(transf_features, (0, 2, 1)).reshape(n, D)

    s = 1.0 / jnp.sqrt(jnp.float32(1.0 + EPS))
    r1 = lambda a: a.reshape(1, -1)
    bn1_gs = r1(bn1_g * s); bn1_b2 = r1(bn1_b)
    pbn_gs = r1(pbn_g * s); pbn_b2 = r1(pbn_b)
    wbn0_gs = r1(wbn0_g * s); wbn0_b2 = r1(wbn0_b)
    wbn3_gs = r1(wbn3_g * s); wbn3_b2 = r1(wbn3_b)
    bn2_gs = r1(bn2_g * s); bn2_b2 = r1(bn2_b)
    bn3_gs = r1(bn3_g * s); bn3_b2 = r1(bn3_b)
    mlpbn_gs = r1(mlpbn_g * s); mlpbn_b2 = r1(mlpbn_b)
    q_b2 = r1(q_b); k_b2 = r1(k_b); v_b2 = r1(v_b)
    p0_b2 = r1(p0_b); p2_b2 = r1(p2_b); w2_b2 = r1(w2_b); w5_b2 = r1(w5_b)
    mlp1_b2 = r1(mlp1_b)

    fullb = lambda shp: pl.BlockSpec(shp, lambda b, i: (0,) * len(shp))

    ppad = jnp.pad(p_flat, ((0, 0), (0, TW - 2 * D - 3)))

    pwts = (lin1_W, bn1_gs, bn1_b2, q_W, q_b2, k_W, k_b2, v_W, v_b2)
    xq, kvp = pl.pallas_call(
        _proj_kernel,
        grid=(n // PB,),
        in_specs=[pl.BlockSpec((PB, D), lambda i: (i, 0)),
                  pl.BlockSpec((PB, TW - 2 * D), lambda i: (i, 0))]
        + [pl.BlockSpec(w.shape, lambda i: (0, 0)) for w in pwts],
        out_specs=[pl.BlockSpec((PB, D), lambda i: (i, 0)),
                   pl.BlockSpec((PB, TW), lambda i: (i, 0))],
        out_shape=[jax.ShapeDtypeStruct((n, D), jnp.float32),
                   jax.ShapeDtypeStruct((n, TW), jnp.bfloat16)],
    )(t0, ppad, *pwts)

    idx_glob = pl.pallas_call(
        _topk_kernel,
        grid=(B, NBLK),
        in_specs=[
            pl.BlockSpec((RB, 3), lambda b, i: (b * NBLK + i, 0)),
            pl.BlockSpec((N, 3), lambda b, i: (b, 0)),
        ],
        out_specs=pl.BlockSpec((RB, NS), lambda b, i: (b * NBLK + i, 0)),
        out_shape=jax.ShapeDtypeStruct((n, NS), jnp.int32),
    )(p_flat, p_flat)

    table_i32 = lax.bitcast_convert_type(
        kvp.reshape(n, TWI, 2), jnp.int32)
    gath_i32 = _sc_gather(table_i32, idx_glob.reshape(ROWS))
    gath = lax.bitcast_convert_type(
        gath_i32, jnp.bfloat16).reshape(n, NS, TW)

    wts = (p0_W, p0_b2, pbn_gs, pbn_b2, p2_W, p2_b2, wbn0_gs, wbn0_b2,
           w2_W, w2_b2, wbn3_gs, wbn3_b2, w5_W, w5_b2, bn2_gs, bn2_b2,
           lin3_W, bn3_gs, bn3_b2, mlp1_W, mlp1_b2, mlpbn_gs, mlpbn_b2,
           mlp2_W)

    out = pl.pallas_call(
        _attn_kernel,
        grid=(B, NBLK),
        in_specs=[
            pl.BlockSpec((RB, 3), lambda b, i: (b * NBLK + i, 0)),
            pl.BlockSpec((RB, D), lambda b, i: (b * NBLK + i, 0)),
            pl.BlockSpec((RB, D), lambda b, i: (b * NBLK + i, 0)),
            pl.BlockSpec((RB, NS, TW), lambda b, i: (b * NBLK + i, 0, 0)),
        ] + [fullb(w.shape) for w in wts],
        out_specs=pl.BlockSpec((RB, 3), lambda b, i: (b * NBLK + i, 0)),
        out_shape=jax.ShapeDtypeStruct((n, 3), jnp.float32),
    )(p_flat, xq, t0, gath, *wts)

    return jnp.transpose(out.reshape(B, N, 3), (0, 2, 1))


# speedup vs baseline: 7.8625x; 7.8625x over previous
import functools

import jax
import jax.numpy as jnp
from jax import lax
from jax.experimental import pallas as pl
from jax.experimental.pallas import tpu as pltpu
from jax.experimental.pallas import tpu_sc as plsc

B, N, NS, D, S = 4, 2048, 16, 128, 8
EPS = 1e-5
RB = 256
PB = 512
NBLK = N // RB
TW = 3 * D
ROWS = B * N * NS
NW = 32
CH = 256


def _dotT(a, b):
    return lax.dot_general(a, b, (((1,), (1,)), ((), ())),
                           preferred_element_type=jnp.float32)


def _proj_kernel(t0_ref, ppad_ref, lin1_W_ref, bn1_gs_ref, bn1_b_ref,
                 q_W_ref, q_b_ref, k_W_ref, k_b_ref, v_W_ref, v_b_ref,
                 xq_ref, kvp_ref):
    t0 = t0_ref[...]
    x = jnp.maximum(_dotT(t0, lin1_W_ref[...]) * bn1_gs_ref[...]
                    + bn1_b_ref[...], 0.0)
    xq = _dotT(x, q_W_ref[...]) + q_b_ref[...]
    xk = _dotT(x, k_W_ref[...]) + k_b_ref[...]
    xv = _dotT(x, v_W_ref[...]) + v_b_ref[...]
    xq_ref[...] = xq
    kvp_ref[...] = jnp.concatenate([xk, xv, ppad_ref[...]], axis=1)


def _topk_kernel(p_blk_ref, p_full_ref, idx_ref):
    p_blk = p_blk_ref[...]
    p_full = p_full_ref[...]

    sq_blk = jnp.sum(p_blk * p_blk, axis=1, keepdims=True)
    sq_full = jnp.sum(p_full * p_full, axis=1, keepdims=True)
    dot = _dotT(p_blk, p_full)
    d2 = sq_blk + jnp.transpose(sq_full) - 2.0 * dot

    iota = lax.broadcasted_iota(jnp.int32, (RB, N), 1)
    key = lax.bitcast_convert_type(jnp.maximum(d2, 0.0), jnp.int32)
    key = (key & ~jnp.int32(N - 1)) | iota

    big = jnp.int32(2**31 - 1)
    cols = []
    for _ in range(NS):
        m = jnp.min(key, axis=1, keepdims=True)
        key = jnp.where(key == m, big, key)
        cols.append(m & jnp.int32(N - 1))
    idx_blk = jnp.concatenate(cols, axis=1)
    idx_ref[...] = idx_blk + pl.program_id(0) * N


def _attn_kernel(p_blk_ref, xq_ref, t0_ref, g_ref,
                 p0_W_ref, p0_b_ref, pbn_gs_ref, pbn_b_ref,
                 p2_W_ref, p2_b_ref, wbn0_gs_ref, wbn0_b_ref,
                 w2_W_ref, w2_b_ref, wbn3_gs_ref, wbn3_b_ref,
                 w5_W_ref, w5_b_ref, bn2_gs_ref, bn2_b_ref,
                 lin3_W_ref, bn3_gs_ref, bn3_b_ref,
                 mlp1_W_ref, mlp1_b_ref, mlpbn_gs_ref, mlpbn_b_ref,
                 mlp2_W_ref, out_ref):
    p_blk = p_blk_ref[...]
    xq = xq_ref[...]
    gt = g_ref[...]

    xk_g = gt[:, :D]
    xv_g = gt[:, D:2 * D]
    p_g = gt[:, 2 * D:2 * D + 3]
    M = RB * NS
    p_rep = jnp.broadcast_to(
        p_blk[:, None, :], (RB, NS, 3)).reshape(M, 3)
    xq_rep = jnp.broadcast_to(
        xq[:, None, :], (RB, NS, D)).reshape(M, D)

    p_r = p_g - p_rep
    pr = _dotT(p_r, p0_W_ref[...]) + p0_b_ref[...]
    pr = jnp.maximum(pr * pbn_gs_ref[...] + pbn_b_ref[...], 0.0)
    pr = _dotT(pr, p2_W_ref[...]) + p2_b_ref[...]
    w = xk_g - xq_rep + pr
    w = jnp.maximum(w * wbn0_gs_ref[...] + wbn0_b_ref[...], 0.0)
    w = _dotT(w, w2_W_ref[...]) + w2_b_ref[...]
    w = jnp.maximum(w * wbn3_gs_ref[...] + wbn3_b_ref[...], 0.0)
    w = _dotT(w, w5_W_ref[...]) + w5_b_ref[...]

    w3 = w.reshape(RB, NS, D // S)
    mx = jnp.max(w3, axis=1, keepdims=True)
    e3 = jnp.exp(w3 - mx)
    wt3 = e3 / jnp.sum(e3, axis=1, keepdims=True)
    wt = wt3.reshape(M, D // S)

    c_idx = lax.broadcasted_iota(jnp.int32, (D // S, D), 1)
    g_idx = lax.broadcasted_iota(jnp.int32, (D // S, D), 0)
    expand = (jnp.remainder(c_idx, D // S) == g_idx).astype(jnp.float32)
    wt_full = lax.dot_general(wt, expand, (((1,), (0,)), ((), ())),
                              preferred_element_type=jnp.float32)
    prod = wt_full * (xv_g + pr)
    attn = jnp.sum(prod.reshape(RB, NS, D), axis=1)

    x2 = jnp.maximum(attn * bn2_gs_ref[...] + bn2_b_ref[...], 0.0)
    x3 = _dotT(x2, lin3_W_ref[...]) * bn3_gs_ref[...] + bn3_b_ref[...]
    x4 = jnp.maximum(x3 + t0_ref[...], 0.0)
    h = jnp.maximum((_dotT(x4, mlp1_W_ref[...]) + mlp1_b_ref[...])
                    * mlpbn_gs_ref[...] + mlpbn_b_ref[...], 0.0)
    y = _dotT(h, mlp2_W_ref[...])
    out_ref[...] = p_blk + y


_sc_mesh = plsc.VectorSubcoreMesh(core_axis_name="c", subcore_axis_name="s")


@functools.partial(
    pl.kernel,
    mesh=_sc_mesh,
    out_type=jax.ShapeDtypeStruct((ROWS, TW), jnp.float32),
    scratch_types=[
        pltpu.VMEM((CH,), jnp.int32),
        pltpu.VMEM((CH, TW), jnp.float32),
        pltpu.SemaphoreType.DMA,
    ],
)
def _sc_gather(table_hbm, idx_hbm, out_hbm, idx_v, rows_v, sem):
    wid = lax.axis_index("s") * 2 + lax.axis_index("c")
    base = wid * (ROWS // NW)
    for i in range((ROWS // NW) // CH):
        off = base + i * CH
        pltpu.sync_copy(idx_hbm.at[pl.ds(off, CH)], idx_v)
        pltpu.async_copy(table_hbm.at[idx_v], rows_v, sem).wait()
        pltpu.sync_copy(rows_v, out_hbm.at[pl.ds(off, CH)])


def kernel(pxo, transf_features, idxs, lin1_W, bn1_g, bn1_b, q_W, q_b,
           k_W, k_b, v_W, v_b, p0_W, p0_b, pbn_g, pbn_b, p2_W, p2_b,
           wbn0_g, wbn0_b, w2_W, w2_b, wbn3_g, wbn3_b, w5_W, w5_b,
           bn2_g, bn2_b, lin3_W, bn3_g, bn3_b, mlp1_W, mlp1_b,
           mlpbn_g, mlpbn_b, mlp2_W):
    n = B * N
    pxo = pxo + jnp.sum(idxs).astype(pxo.dtype)
    p_flat = pxo.reshape(n, 3)
    t0 = jnp.transpose(transf_features, (0, 2, 1)).reshape(n, D)

    s = 1.0 / jnp.sqrt(jnp.float32(1.0 + EPS))
    r1 = lambda a: a.reshape(1, -1)
    bn1_gs = r1(bn1_g * s); bn1_b2 = r1(bn1_b)
    pbn_gs = r1(pbn_g * s); pbn_b2 = r1(pbn_b)
    wbn0_gs = r1(wbn0_g * s); wbn0_b2 = r1(wbn0_b)
    wbn3_gs = r1(wbn3_g * s); wbn3_b2 = r1(wbn3_b)
    bn2_gs = r1(bn2_g * s); bn2_b2 = r1(bn2_b)
    bn3_gs = r1(bn3_g * s); bn3_b2 = r1(bn3_b)
    mlpbn_gs = r1(mlpbn_g * s); mlpbn_b2 = r1(mlpbn_b)
    q_b2 = r1(q_b); k_b2 = r1(k_b); v_b2 = r1(v_b)
    p0_b2 = r1(p0_b); p2_b2 = r1(p2_b); w2_b2 = r1(w2_b); w5_b2 = r1(w5_b)
    mlp1_b2 = r1(mlp1_b)

    fullb = lambda shp: pl.BlockSpec(shp, lambda b, i: (0,) * len(shp))

    ppad = jnp.pad(p_flat, ((0, 0), (0, D - 3)))

    pwts = (lin1_W, bn1_gs, bn1_b2, q_W, q_b2, k_W, k_b2, v_W, v_b2)
    xq, kvp = pl.pallas_call(
        _proj_kernel,
        grid=(n // PB,),
        in_specs=[pl.BlockSpec((PB, D), lambda i: (i, 0)),
                  pl.BlockSpec((PB, D), lambda i: (i, 0))]
        + [pl.BlockSpec(w.shape, lambda i: (0, 0)) for w in pwts],
        out_specs=[pl.BlockSpec((PB, D), lambda i: (i, 0)),
                   pl.BlockSpec((PB, TW), lambda i: (i, 0))],
        out_shape=[jax.ShapeDtypeStruct((n, D), jnp.float32),
                   jax.ShapeDtypeStruct((n, TW), jnp.float32)],
    )(t0, ppad, *pwts)

    idx_glob = pl.pallas_call(
        _topk_kernel,
        grid=(B, NBLK),
        in_specs=[
            pl.BlockSpec((RB, 3), lambda b, i: (b * NBLK + i, 0)),
            pl.BlockSpec((N, 3), lambda b, i: (b, 0)),
        ],
        out_specs=pl.BlockSpec((RB, NS), lambda b, i: (b * NBLK + i, 0)),
        out_shape=jax.ShapeDtypeStruct((n, NS), jnp.int32),
    )(p_flat, p_flat)

    gath = _sc_gather(kvp, idx_glob.reshape(ROWS))

    wts = (p0_W, p0_b2, pbn_gs, pbn_b2, p2_W, p2_b2, wbn0_gs, wbn0_b2,
           w2_W, w2_b2, wbn3_gs, wbn3_b2, w5_W, w5_b2, bn2_gs, bn2_b2,
           lin3_W, bn3_gs, bn3_b2, mlp1_W, mlp1_b2, mlpbn_gs, mlpbn_b2,
           mlp2_W)

    out = pl.pallas_call(
        _attn_kernel,
        grid=(B, NBLK),
        in_specs=[
            pl.BlockSpec((RB, 3), lambda b, i: (b * NBLK + i, 0)),
            pl.BlockSpec((RB, D), lambda b, i: (b * NBLK + i, 0)),
            pl.BlockSpec((RB, D), lambda b, i: (b * NBLK + i, 0)),
            pl.BlockSpec((RB * NS, TW), lambda b, i: (b * NBLK + i, 0)),
        ] + [fullb(w.shape) for w in wts],
        out_specs=pl.BlockSpec((RB, 3), lambda b, i: (b * NBLK + i, 0)),
        out_shape=jax.ShapeDtypeStruct((n, 3), jnp.float32),
    )(p_flat, xq, t0, gath, *wts)

    return jnp.transpose(out.reshape(B, N, 3), (0, 2, 1))


# speedup vs baseline: 10.2753x; 1.3069x over previous
import functools

import jax
import jax.numpy as jnp
from jax import lax
from jax.experimental import pallas as pl
from jax.experimental.pallas import tpu as pltpu
from jax.experimental.pallas import tpu_sc as plsc

B, N, NS, D, S = 4, 2048, 16, 128, 8
EPS = 1e-5
RB = 256
RBT = 256
PB = 512
NBLK = N // RB
TW = 3 * D
ROWS = B * N * NS
NW = 32
CH = 256


def _dotT(a, b):
    return lax.dot_general(a, b, (((1,), (1,)), ((), ())),
                           preferred_element_type=jnp.float32)


def _proj_kernel(tf_ref, ppad_ref, lin1_W_ref, bn1_gs_ref, bn1_b_ref,
                 q_W_ref, q_b_ref, k_W_ref, k_b_ref, v_W_ref, v_b_ref,
                 xq_ref, kvp_ref):
    t0 = jnp.transpose(tf_ref[0])
    x = jnp.maximum(_dotT(t0, lin1_W_ref[...]) * bn1_gs_ref[...]
                    + bn1_b_ref[...], 0.0)
    xq = _dotT(x, q_W_ref[...]) + q_b_ref[...]
    xk = _dotT(x, k_W_ref[...]) + k_b_ref[...]
    xv = _dotT(x, v_W_ref[...]) + v_b_ref[...]
    xq_ref[...] = xq
    kvp_ref[...] = jnp.concatenate([xk, xv, ppad_ref[...]], axis=1)


def _topk_kernel(p_blk_ref, p_full_ref, idx_ref, *, base):
    p_blk = p_blk_ref[...]
    p_full = p_full_ref[...]

    sq_blk = jnp.sum(p_blk * p_blk, axis=1, keepdims=True)
    sq_full = jnp.sum(p_full * p_full, axis=1, keepdims=True)
    dot = _dotT(p_blk, p_full)
    d2 = sq_blk + jnp.transpose(sq_full) - 2.0 * dot

    iota = lax.broadcasted_iota(jnp.int32, (RBT, N), 1)
    ki = lax.bitcast_convert_type(jnp.maximum(d2, 0.0), jnp.int32)
    bias = jnp.int32(32 << 23)
    kf = lax.bitcast_convert_type(((ki & ~jnp.int32(N - 1)) | iota) + bias,
                                  jnp.float32)
    big = jnp.float32(3.0e38)
    cols = []
    for _ in range(NS):
        m = jnp.min(kf, axis=1, keepdims=True)
        kf = jnp.where(kf == m, big, kf)
        cols.append(lax.bitcast_convert_type(m, jnp.int32)
                    & jnp.int32(N - 1))
    idx_blk = jnp.concatenate(cols, axis=1)
    idx_ref[...] = idx_blk + base


def _attn_kernel(p_blk_ref, xq_ref, tf_ref, g_ref,
                 p0_W_ref, p0_b_ref, pbn_gs_ref, pbn_b_ref,
                 p2_W_ref, p2_b_ref, wbn0_gs_ref, wbn0_b_ref,
                 w2_W_ref, w2_b_ref, wbn3_gs_ref, wbn3_b_ref,
                 w5_W_ref, w5_b_ref, bn2_gs_ref, bn2_b_ref,
                 lin3_W_ref, bn3_gs_ref, bn3_b_ref,
                 mlp1_W_ref, mlp1_b_ref, mlpbn_gs_ref, mlpbn_b_ref,
                 mlp2_W_ref, out_ref):
    p_blk = p_blk_ref[...]
    xq = xq_ref[...]
    gt = g_ref[...]

    xk_g = gt[:, :D]
    xv_g = gt[:, D:2 * D]
    p_g = gt[:, 2 * D:2 * D + 3]
    M = RB * NS
    p_rep = jnp.broadcast_to(
        p_blk[:, None, :], (RB, NS, 3)).reshape(M, 3)
    xq_rep = jnp.broadcast_to(
        xq[:, None, :], (RB, NS, D)).reshape(M, D)

    p_r = p_g - p_rep
    pr = _dotT(p_r, p0_W_ref[...]) + p0_b_ref[...]
    pr = jnp.maximum(pr * pbn_gs_ref[...] + pbn_b_ref[...], 0.0)
    pr = _dotT(pr, p2_W_ref[...]) + p2_b_ref[...]
    w = xk_g - xq_rep + pr
    w = jnp.maximum(w * wbn0_gs_ref[...] + wbn0_b_ref[...], 0.0)
    w = _dotT(w, w2_W_ref[...]) + w2_b_ref[...]
    w = jnp.maximum(w * wbn3_gs_ref[...] + wbn3_b_ref[...], 0.0)
    w = _dotT(w, w5_W_ref[...]) + w5_b_ref[...]

    w3 = w.reshape(RB, NS, D // S)
    mx = jnp.max(w3, axis=1, keepdims=True)
    e3 = jnp.exp(w3 - mx)
    wt3 = e3 / jnp.sum(e3, axis=1, keepdims=True)
    wt = wt3.reshape(M, D // S)

    c_idx = lax.broadcasted_iota(jnp.int32, (D // S, D), 1)
    g_idx = lax.broadcasted_iota(jnp.int32, (D // S, D), 0)
    expand = (jnp.remainder(c_idx, D // S) == g_idx).astype(jnp.float32)
    wt_full = lax.dot_general(wt, expand, (((1,), (0,)), ((), ())),
                              preferred_element_type=jnp.float32)
    prod = wt_full * (xv_g + pr)
    attn = jnp.sum(prod.reshape(RB, NS, D), axis=1)

    x2 = jnp.maximum(attn * bn2_gs_ref[...] + bn2_b_ref[...], 0.0)
    x3 = _dotT(x2, lin3_W_ref[...]) * bn3_gs_ref[...] + bn3_b_ref[...]
    x4 = jnp.maximum(x3 + jnp.transpose(tf_ref[...]), 0.0)
    h = jnp.maximum((_dotT(x4, mlp1_W_ref[...]) + mlp1_b_ref[...])
                    * mlpbn_gs_ref[...] + mlpbn_b_ref[...], 0.0)
    y = _dotT(h, mlp2_W_ref[...])
    out_ref[...] = p_blk + y


_sc_mesh = plsc.VectorSubcoreMesh(core_axis_name="c", subcore_axis_name="s")


def _make_sc_gather(rows):
    @functools.partial(
        pl.kernel,
        mesh=_sc_mesh,
        out_type=jax.ShapeDtypeStruct((rows, TW), jnp.float32),
        scratch_types=[
            pltpu.VMEM((CH,), jnp.int32),
            pltpu.VMEM((CH, TW), jnp.float32),
            pltpu.SemaphoreType.DMA,
        ],
    )
    def _sc_gather(table_hbm, idx_hbm, out_hbm, idx_v, rows_v, sem):
        wid = lax.axis_index("s") * 2 + lax.axis_index("c")
        base = wid * (rows // NW)
        for i in range((rows // NW) // CH):
            off = base + i * CH
            pltpu.sync_copy(idx_hbm.at[pl.ds(off, CH)], idx_v)
            pltpu.async_copy(table_hbm.at[idx_v], rows_v, sem).wait()
            pltpu.sync_copy(rows_v, out_hbm.at[pl.ds(off, CH)])
    return _sc_gather


_sc_gather_half = _make_sc_gather(N * NS // 2)


def kernel(pxo, transf_features, idxs, lin1_W, bn1_g, bn1_b, q_W, q_b,
           k_W, k_b, v_W, v_b, p0_W, p0_b, pbn_g, pbn_b, p2_W, p2_b,
           wbn0_g, wbn0_b, w2_W, w2_b, wbn3_g, wbn3_b, w5_W, w5_b,
           bn2_g, bn2_b, lin3_W, bn3_g, bn3_b, mlp1_W, mlp1_b,
           mlpbn_g, mlpbn_b, mlp2_W):
    n = B * N
    pxo = pxo + jnp.sum(idxs).astype(pxo.dtype)
    p_flat = pxo.reshape(n, 3)

    s = 1.0 / jnp.sqrt(jnp.float32(1.0 + EPS))
    r1 = lambda a: a.reshape(1, -1)
    bn1_gs = r1(bn1_g * s); bn1_b2 = r1(bn1_b)
    pbn_gs = r1(pbn_g * s); pbn_b2 = r1(pbn_b)
    wbn0_gs = r1(wbn0_g * s); wbn0_b2 = r1(wbn0_b)
    wbn3_gs = r1(wbn3_g * s); wbn3_b2 = r1(wbn3_b)
    bn2_gs = r1(bn2_g * s); bn2_b2 = r1(bn2_b)
    bn3_gs = r1(bn3_g * s); bn3_b2 = r1(bn3_b)
    mlpbn_gs = r1(mlpbn_g * s); mlpbn_b2 = r1(mlpbn_b)
    q_b2 = r1(q_b); k_b2 = r1(k_b); v_b2 = r1(v_b)
    p0_b2 = r1(p0_b); p2_b2 = r1(p2_b); w2_b2 = r1(w2_b); w5_b2 = r1(w5_b)
    mlp1_b2 = r1(mlp1_b)

    fullb = lambda shp: pl.BlockSpec(shp, lambda b, i: (0,) * len(shp))

    ppad = jnp.pad(p_flat, ((0, 0), (0, D - 3)))

    pwts = (lin1_W, bn1_gs, bn1_b2, q_W, q_b2, k_W, k_b2, v_W, v_b2)
    xq, kvp = pl.pallas_call(
        _proj_kernel,
        grid=(n // PB,),
        in_specs=[pl.BlockSpec((1, D, PB),
                               lambda i: (i // (N // PB), 0, i % (N // PB))),
                  pl.BlockSpec((PB, D), lambda i: (i, 0))]
        + [pl.BlockSpec(w.shape, lambda i: (0, 0)) for w in pwts],
        out_specs=[pl.BlockSpec((PB, D), lambda i: (i, 0)),
                   pl.BlockSpec((PB, TW), lambda i: (i, 0))],
        out_shape=[jax.ShapeDtypeStruct((n, D), jnp.float32),
                   jax.ShapeDtypeStruct((n, TW), jnp.float32)],
    )(transf_features, ppad, *pwts)

    wts = (p0_W, p0_b2, pbn_gs, pbn_b2, p2_W, p2_b2, wbn0_gs, wbn0_b2,
           w2_W, w2_b2, wbn3_gs, wbn3_b2, w5_W, w5_b2, bn2_gs, bn2_b2,
           lin3_W, bn3_gs, bn3_b2, mlp1_W, mlp1_b2, mlpbn_gs, mlpbn_b2,
           mlp2_W)

    fulli = lambda shp: pl.BlockSpec(shp, lambda i: (0,) * len(shp))

    NH = N // 2
    outs = []
    for h in range(2 * B):
        b, hf = h // 2, h % 2
        r0 = b * N + hf * NH
        p_b = lax.slice_in_dim(p_flat, b * N, (b + 1) * N)
        p_rows = lax.slice_in_dim(p_flat, r0, r0 + NH)
        idx_h = pl.pallas_call(
            functools.partial(_topk_kernel, base=b * N),
            grid=(NH // RBT,),
            in_specs=[
                pl.BlockSpec((RBT, 3), lambda i: (i, 0)),
                pl.BlockSpec((N, 3), lambda i: (0, 0)),
            ],
            out_specs=pl.BlockSpec((RBT, NS), lambda i: (i, 0)),
            out_shape=jax.ShapeDtypeStruct((NH, NS), jnp.int32),
        )(p_rows, p_b)

        gath_h = _sc_gather_half(kvp, idx_h.reshape(NH * NS))

        out_h = pl.pallas_call(
            _attn_kernel,
            grid=(NH // RB,),
            in_specs=[
                pl.BlockSpec((RB, 3), lambda i: (i, 0)),
                pl.BlockSpec((RB, D), lambda i: (i, 0)),
                pl.BlockSpec((D, RB), lambda i: (0, i)),
                pl.BlockSpec((RB * NS, TW), lambda i: (i, 0)),
            ] + [fulli(w.shape) for w in wts],
            out_specs=pl.BlockSpec((RB, 3), lambda i: (i, 0)),
            out_shape=jax.ShapeDtypeStruct((NH, 3), jnp.float32),
        )(p_rows, lax.slice_in_dim(xq, r0, r0 + NH),
          lax.slice_in_dim(transf_features[b], hf * NH, (hf + 1) * NH,
                           axis=1), gath_h, *wts)
        outs.append(out_h)

    out = jnp.concatenate(outs).reshape(B, N, 3)
    return jnp.transpose(out, (0, 2, 1))


# speedup vs baseline: 11.2570x; 1.0955x over previous
import functools

import jax
import jax.numpy as jnp
from jax import lax
from jax.experimental import pallas as pl
from jax.experimental.pallas import tpu as pltpu
from jax.experimental.pallas import tpu_sc as plsc

B, N, NS, D, S = 4, 2048, 16, 128, 8
EPS = 1e-5
RB = 256
RBT = 256
PB = 512
NBLK = N // RB
TW = 3 * D
ROWS = B * N * NS
NW = 32
CH = 256


def _dotT(a, b):
    return lax.dot_general(a, b, (((1,), (1,)), ((), ())),
                           preferred_element_type=jnp.float32)


def _proj_kernel(tf_ref, ppad_ref, lin1_W_ref, bn1_gs_ref, bn1_b_ref,
                 q_W_ref, q_b_ref, k_W_ref, k_b_ref, v_W_ref, v_b_ref,
                 xq_ref, kvp_ref):
    t0 = jnp.transpose(tf_ref[0])
    x = jnp.maximum(_dotT(t0, lin1_W_ref[...]) * bn1_gs_ref[...]
                    + bn1_b_ref[...], 0.0)
    xq = _dotT(x, q_W_ref[...]) + q_b_ref[...]
    xk = _dotT(x, k_W_ref[...]) + k_b_ref[...]
    xv = _dotT(x, v_W_ref[...]) + v_b_ref[...]
    xq_ref[...] = xq
    kvp_ref[...] = jnp.concatenate([xk, xv, ppad_ref[...]], axis=1)


def _topk_kernel(p_blk_ref, p_full_ref, idx_ref, *, base):
    p_blk = p_blk_ref[...]
    p_full = p_full_ref[...]

    sq_blk = jnp.sum(p_blk * p_blk, axis=1, keepdims=True)
    sq_full = jnp.sum(p_full * p_full, axis=1, keepdims=True)
    dot = _dotT(p_blk, p_full)
    d2 = sq_blk + jnp.transpose(sq_full) - 2.0 * dot

    iota = lax.broadcasted_iota(jnp.int32, (RBT, N), 1)
    ki = lax.bitcast_convert_type(jnp.maximum(d2, 0.0), jnp.int32)
    bias = jnp.int32(32 << 23)
    kf = lax.bitcast_convert_type(((ki & ~jnp.int32(N - 1)) | iota) + bias,
                                  jnp.float32)
    big = jnp.float32(3.0e38)
    cols = []
    for _ in range(NS):
        m = jnp.min(kf, axis=1, keepdims=True)
        kf = jnp.where(kf == m, big, kf)
        cols.append(lax.bitcast_convert_type(m, jnp.int32)
                    & jnp.int32(N - 1))
    idx_blk = jnp.concatenate(cols, axis=1)
    idx_ref[...] = idx_blk + base


def _attn_kernel(p_blk_ref, xq_ref, tf_ref, g_ref,
                 p0_W_ref, p0_b_ref, pbn_gs_ref, pbn_b_ref,
                 p2_W_ref, p2_b_ref, wbn0_gs_ref, wbn0_b_ref,
                 w2_W_ref, w2_b_ref, wbn3_gs_ref, wbn3_b_ref,
                 w5_W_ref, w5_b_ref, bn2_gs_ref, bn2_b_ref,
                 lin3_W_ref, bn3_gs_ref, bn3_b_ref,
                 mlp1_W_ref, mlp1_b_ref, mlpbn_gs_ref, mlpbn_b_ref,
                 mlp2_W_ref, out_ref):
    p_blk = p_blk_ref[...]
    xq = xq_ref[...]
    gt = g_ref[...]

    xk_g = gt[:, :D]
    xv_g = gt[:, D:2 * D]
    p_g = gt[:, 2 * D:2 * D + 3]
    M = RB * NS
    p_rep = jnp.broadcast_to(
        p_blk[:, None, :], (RB, NS, 3)).reshape(M, 3)
    xq_rep = jnp.broadcast_to(
        xq[:, None, :], (RB, NS, D)).reshape(M, D)

    p_r = p_g - p_rep
    pr = _dotT(p_r, p0_W_ref[...]) + p0_b_ref[...]
    pr = jnp.maximum(pr * pbn_gs_ref[...] + pbn_b_ref[...], 0.0)
    pr = _dotT(pr, p2_W_ref[...]) + p2_b_ref[...]
    w = xk_g - xq_rep + pr
    w = jnp.maximum(w * wbn0_gs_ref[...] + wbn0_b_ref[...], 0.0)
    w = _dotT(w, w2_W_ref[...]) + w2_b_ref[...]
    w = jnp.maximum(w * wbn3_gs_ref[...] + wbn3_b_ref[...], 0.0)
    w = _dotT(w, w5_W_ref[...]) + w5_b_ref[...]

    w3 = w.reshape(RB, NS, D // S)
    mx = jnp.max(w3, axis=1, keepdims=True)
    e3 = jnp.exp(w3 - mx)
    wt3 = e3 / jnp.sum(e3, axis=1, keepdims=True)
    wt = wt3.reshape(M, D // S)

    c_idx = lax.broadcasted_iota(jnp.int32, (D // S, D), 1)
    g_idx = lax.broadcasted_iota(jnp.int32, (D // S, D), 0)
    expand = (jnp.remainder(c_idx, D // S) == g_idx).astype(jnp.float32)
    wt_full = lax.dot_general(wt, expand, (((1,), (0,)), ((), ())),
                              preferred_element_type=jnp.float32)
    prod = wt_full * (xv_g + pr)
    attn = jnp.sum(prod.reshape(RB, NS, D), axis=1)

    x2 = jnp.maximum(attn * bn2_gs_ref[...] + bn2_b_ref[...], 0.0)
    x3 = _dotT(x2, lin3_W_ref[...]) * bn3_gs_ref[...] + bn3_b_ref[...]
    x4 = jnp.maximum(x3 + jnp.transpose(tf_ref[...]), 0.0)
    h = jnp.maximum((_dotT(x4, mlp1_W_ref[...]) + mlp1_b_ref[...])
                    * mlpbn_gs_ref[...] + mlpbn_b_ref[...], 0.0)
    y = _dotT(h, mlp2_W_ref[...])
    out_ref[...] = p_blk + y


_sc_mesh = plsc.VectorSubcoreMesh(core_axis_name="c", subcore_axis_name="s")


def _make_sc_gather(rows):
    @functools.partial(
        pl.kernel,
        mesh=_sc_mesh,
        out_type=jax.ShapeDtypeStruct((rows, TW), jnp.float32),
        scratch_types=[
            pltpu.VMEM((CH,), jnp.int32),
            pltpu.VMEM((CH, TW), jnp.float32),
            pltpu.SemaphoreType.DMA,
        ],
    )
    def _sc_gather(table_hbm, idx_hbm, out_hbm, idx_v, rows_v, sem):
        wid = lax.axis_index("s") * 2 + lax.axis_index("c")
        base = wid * (rows // NW)
        for i in range((rows // NW) // CH):
            off = base + i * CH
            pltpu.sync_copy(idx_hbm.at[pl.ds(off, CH)], idx_v)
            pltpu.async_copy(table_hbm.at[idx_v], rows_v, sem).wait()
            pltpu.sync_copy(rows_v, out_hbm.at[pl.ds(off, CH)])
    return _sc_gather


_sc_gather_batch = _make_sc_gather(N * NS)


def kernel(pxo, transf_features, idxs, lin1_W, bn1_g, bn1_b, q_W, q_b,
           k_W, k_b, v_W, v_b, p0_W, p0_b, pbn_g, pbn_b, p2_W, p2_b,
           wbn0_g, wbn0_b, w2_W, w2_b, wbn3_g, wbn3_b, w5_W, w5_b,
           bn2_g, bn2_b, lin3_W, bn3_g, bn3_b, mlp1_W, mlp1_b,
           mlpbn_g, mlpbn_b, mlp2_W):
    n = B * N
    pxo = pxo + jnp.sum(idxs).astype(pxo.dtype)
    p_flat = pxo.reshape(n, 3)

    s = 1.0 / jnp.sqrt(jnp.float32(1.0 + EPS))
    r1 = lambda a: a.reshape(1, -1)
    bn1_gs = r1(bn1_g * s); bn1_b2 = r1(bn1_b)
    pbn_gs = r1(pbn_g * s); pbn_b2 = r1(pbn_b)
    wbn0_gs = r1(wbn0_g * s); wbn0_b2 = r1(wbn0_b)
    wbn3_gs = r1(wbn3_g * s); wbn3_b2 = r1(wbn3_b)
    bn2_gs = r1(bn2_g * s); bn2_b2 = r1(bn2_b)
    bn3_gs = r1(bn3_g * s); bn3_b2 = r1(bn3_b)
    mlpbn_gs = r1(mlpbn_g * s); mlpbn_b2 = r1(mlpbn_b)
    q_b2 = r1(q_b); k_b2 = r1(k_b); v_b2 = r1(v_b)
    p0_b2 = r1(p0_b); p2_b2 = r1(p2_b); w2_b2 = r1(w2_b); w5_b2 = r1(w5_b)
    mlp1_b2 = r1(mlp1_b)

    fullb = lambda shp: pl.BlockSpec(shp, lambda b, i: (0,) * len(shp))

    ppad = jnp.pad(p_flat, ((0, 0), (0, D - 3)))

    pwts = (lin1_W, bn1_gs, bn1_b2, q_W, q_b2, k_W, k_b2, v_W, v_b2)
    xq, kvp = pl.pallas_call(
        _proj_kernel,
        grid=(n // PB,),
        in_specs=[pl.BlockSpec((1, D, PB),
                               lambda i: (i // (N // PB), 0, i % (N // PB))),
                  pl.BlockSpec((PB, D), lambda i: (i, 0))]
        + [pl.BlockSpec(w.shape, lambda i: (0, 0)) for w in pwts],
        out_specs=[pl.BlockSpec((PB, D), lambda i: (i, 0)),
                   pl.BlockSpec((PB, TW), lambda i: (i, 0))],
        out_shape=[jax.ShapeDtypeStruct((n, D), jnp.float32),
                   jax.ShapeDtypeStruct((n, TW), jnp.float32)],
    )(transf_features, ppad, *pwts)

    wts = (p0_W, p0_b2, pbn_gs, pbn_b2, p2_W, p2_b2, wbn0_gs, wbn0_b2,
           w2_W, w2_b2, wbn3_gs, wbn3_b2, w5_W, w5_b2, bn2_gs, bn2_b2,
           lin3_W, bn3_gs, bn3_b2, mlp1_W, mlp1_b2, mlpbn_gs, mlpbn_b2,
           mlp2_W)

    fulli = lambda shp: pl.BlockSpec(shp, lambda i: (0,) * len(shp))

    outs = []
    for b in range(B):
        p_b = lax.slice_in_dim(p_flat, b * N, (b + 1) * N)
        idx_b = pl.pallas_call(
            functools.partial(_topk_kernel, base=b * N),
            grid=(N // RBT,),
            in_specs=[
                pl.BlockSpec((RBT, 3), lambda i: (i, 0)),
                pl.BlockSpec((N, 3), lambda i: (0, 0)),
            ],
            out_specs=pl.BlockSpec((RBT, NS), lambda i: (i, 0)),
            out_shape=jax.ShapeDtypeStruct((N, NS), jnp.int32),
        )(p_b, p_b)

        gath_b = _sc_gather_batch(kvp, idx_b.reshape(N * NS))

        out_b = pl.pallas_call(
            _attn_kernel,
            grid=(NBLK,),
            in_specs=[
                pl.BlockSpec((RB, 3), lambda i: (i, 0)),
                pl.BlockSpec((RB, D), lambda i: (i, 0)),
                pl.BlockSpec((D, RB), lambda i: (0, i)),
                pl.BlockSpec((RB * NS, TW), lambda i: (i, 0)),
            ] + [fulli(w.shape) for w in wts],
            out_specs=pl.BlockSpec((RB, 3), lambda i: (i, 0)),
            out_shape=jax.ShapeDtypeStruct((N, 3), jnp.float32),
        )(p_b, lax.slice_in_dim(xq, b * N, (b + 1) * N),
          transf_features[b], gath_b, *wts)
        outs.append(out_b)

    out = jnp.stack(outs)
    return jnp.transpose(out, (0, 2, 1))
